# baseline (speedup 1.0000x reference)
import functools

import jax
import jax.numpy as jnp
from jax import lax
from jax.experimental import pallas as pl
from jax.experimental.pallas import tpu as pltpu
from jax.experimental.pallas import tpu_sc as plsc

NC = 2
NS = 16
NW = NC * NS


def _sc_gather(user_ids, item_ids, gamma_user, gamma_item, theta_user,
               visual_features):
    B = user_ids.shape[0]
    DG = gamma_user.shape[1]
    DV = visual_features.shape[1]
    bpw = B // NW
    CH = 64
    nch = bpw // CH

    mesh = plsc.VectorSubcoreMesh(core_axis_name="c", subcore_axis_name="s",
                                  num_cores=NC, num_subcores=NS)

    @functools.partial(
        pl.kernel,
        out_type=[
            jax.ShapeDtypeStruct((B, DG), jnp.float32),
            jax.ShapeDtypeStruct((B, DG), jnp.float32),
            jax.ShapeDtypeStruct((B, DG), jnp.float32),
            jax.ShapeDtypeStruct((B, DV), jnp.float32),
        ],
        mesh=mesh,
        compiler_params=pltpu.CompilerParams(use_tc_tiling_on_sc=False),
        scratch_types=[
            pltpu.VMEM((bpw,), jnp.int32),
            pltpu.VMEM((bpw,), jnp.int32),
            pltpu.VMEM((bpw, DG), jnp.float32),
            pltpu.VMEM((bpw, DG), jnp.float32),
            pltpu.VMEM((bpw, DG), jnp.float32),
            pltpu.VMEM((CH, DV), jnp.float32),
            pltpu.VMEM((CH, DV), jnp.float32),
            pltpu.SemaphoreType.DMA,
            pltpu.SemaphoreType.DMA,
            pltpu.SemaphoreType.DMA,
            pltpu.SemaphoreType.DMA,
            pltpu.SemaphoreType.DMA,
        ],
    )
    def sc_k(uid_h, iid_h, gu_h, gi_h, tu_h, vf_h,
             gu_o, gi_o, tu_o, vf_o,
             idx_u, idx_i, gu_v, gi_v, tu_v, va_v, vb_v,
             s_gu, s_gi, s_tu, s_a, s_b):
        wid = lax.axis_index("s") * NC + lax.axis_index("c")
        base = wid * bpw
        pltpu.sync_copy(uid_h.at[pl.ds(base, bpw)], idx_u)
        pltpu.sync_copy(iid_h.at[pl.ds(base, bpw)], idx_i)
        pltpu.async_copy(gu_h.at[idx_u], gu_v, s_gu)
        pltpu.async_copy(gi_h.at[idx_i], gi_v, s_gi)
        pltpu.async_copy(tu_h.at[idx_u], tu_v, s_tu)
        bufs = (va_v, vb_v)
        sems = (s_a, s_b)
        pltpu.async_copy(vf_h.at[idx_i.at[pl.ds(0, CH)]], bufs[0], sems[0])
        for c in range(nch):
            cur, s_cur = bufs[c % 2], sems[c % 2]
            pltpu.make_async_copy(
                vf_h.at[idx_i.at[pl.ds(c * CH, CH)]], cur, s_cur).wait()
            if c + 1 < nch:
                nxt, s_nxt = bufs[(c + 1) % 2], sems[(c + 1) % 2]
                pltpu.async_copy(
                    vf_h.at[idx_i.at[pl.ds((c + 1) * CH, CH)]], nxt, s_nxt)
            pltpu.sync_copy(cur, vf_o.at[pl.ds(base + c * CH, CH)])
        pltpu.make_async_copy(gu_h.at[idx_u], gu_v, s_gu).wait()
        pltpu.sync_copy(gu_v, gu_o.at[pl.ds(base, bpw)])
        pltpu.make_async_copy(gi_h.at[idx_i], gi_v, s_gi).wait()
        pltpu.sync_copy(gi_v, gi_o.at[pl.ds(base, bpw)])
        pltpu.make_async_copy(tu_h.at[idx_u], tu_v, s_tu).wait()
        pltpu.sync_copy(tu_v, tu_o.at[pl.ds(base, bpw)])

    return sc_k(user_ids, item_ids, gamma_user, gamma_item, theta_user,
                visual_features)


def _tc_score(gu_g, gi_g, tu_g, vis_g, E_w):
    B = gu_g.shape[0]
    DG = gu_g.shape[1]
    DV = vis_g.shape[1]
    BLK = 1024
    nblk = B // BLK

    def body(gu_r, gi_r, tu_r, vis_r, ew_r, out_r):
        proj = lax.dot_general(
            vis_r[...], ew_r[...], (((1,), (1,)), ((), ())),
            preferred_element_type=jnp.float32,
            precision=lax.Precision.HIGHEST)
        cf = jnp.sum(gu_r[...] * gi_r[...], axis=1)
        vs = jnp.sum(tu_r[...] * proj, axis=1)
        out_r[...] = (cf + vs).reshape(BLK // 128, 128)

    out2d = pl.pallas_call(
        body,
        grid=(nblk,),
        in_specs=[
            pl.BlockSpec((BLK, DG), lambda i: (i, 0)),
            pl.BlockSpec((BLK, DG), lambda i: (i, 0)),
            pl.BlockSpec((BLK, DG), lambda i: (i, 0)),
            pl.BlockSpec((BLK, DV), lambda i: (i, 0)),
            pl.BlockSpec((DG, DV), lambda i: (0, 0)),
        ],
        out_specs=pl.BlockSpec((BLK // 128, 128), lambda i: (i, 0)),
        out_shape=jax.ShapeDtypeStruct((B // 128, 128), jnp.float32),
    )(gu_g, gi_g, tu_g, vis_g, E_w)
    return out2d.reshape(B)


def kernel(user_ids, item_ids, gamma_user, gamma_item, theta_user, E_w,
           visual_features):
    gu_g, gi_g, tu_g, vis_g = _sc_gather(
        user_ids, item_ids, gamma_user, gamma_item, theta_user,
        visual_features)
    return _tc_score(gu_g, gi_g, tu_g, vis_g, E_w)


# speedup vs baseline: 1.0771x; 1.0771x over previous
import functools

import jax
import jax.numpy as jnp
from jax import lax
from jax.experimental import pallas as pl
from jax.experimental.pallas import tpu as pltpu
from jax.experimental.pallas import tpu_sc as plsc

NC = 2
NS = 16
NW = NC * NS
LANES = 16


def _sc_gather(user_ids, item_ids, gamma_user4, gamma_item4, theta_user4,
               visual_features):
    B = user_ids.shape[0]
    DG = 32
    DV = visual_features.shape[1]
    bpw = B // NW
    CH = 16
    nch = bpw // CH
    HALF = bpw // 2
    ng = HALF // LANES

    mesh = plsc.VectorSubcoreMesh(core_axis_name="c", subcore_axis_name="s",
                                  num_cores=NC, num_subcores=NS)

    @functools.partial(
        pl.kernel,
        out_type=[
            jax.ShapeDtypeStruct((B * DG,), jnp.float32),
            jax.ShapeDtypeStruct((B, DV), jnp.float32),
            jax.ShapeDtypeStruct((B,), jnp.float32),
        ],
        mesh=mesh,
        compiler_params=pltpu.CompilerParams(needs_layout_passes=False),
        scratch_types=[
            pltpu.VMEM((bpw,), jnp.int32),
            pltpu.VMEM((bpw,), jnp.int32),
            pltpu.VMEM((bpw,), jnp.int32),
            pltpu.VMEM((bpw,), jnp.int32),
            pltpu.VMEM((HALF, 128), jnp.float32),
            pltpu.VMEM((HALF, 128), jnp.float32),
            pltpu.VMEM((bpw * DG,), jnp.float32),
            pltpu.VMEM((bpw * DG,), jnp.float32),
            pltpu.VMEM((bpw,), jnp.float32),
            pltpu.VMEM((CH, DV), jnp.float32),
            pltpu.VMEM((CH, DV), jnp.float32),
            pltpu.SemaphoreType.DMA,
            pltpu.SemaphoreType.DMA,
            pltpu.SemaphoreType.DMA,
            pltpu.SemaphoreType.DMA,
        ],
    )
    def sc_k(uid_h, iid_h, gu4_h, gi4_h, tu4_h, vf_h,
             tu_o, vf_o, cf_o,
             idx_u, idx_i, idx_u4, idx_i4, wa_v, wb_v, gu_v, tu_v, cf_v,
             va_v, vb_v, s_wa, s_wb, s_a, s_b):
        wid = lax.axis_index("s") * NC + lax.axis_index("c")
        base = wid * bpw
        pltpu.sync_copy(uid_h.at[pl.ds(base, bpw)], idx_u)
        pltpu.sync_copy(iid_h.at[pl.ds(base, bpw)], idx_i)
        for j in range(bpw // LANES):
            sl = pl.ds(j * LANES, LANES)
            idx_u4[sl] = lax.shift_right_logical(idx_u[sl], 2)
            idx_i4[sl] = lax.shift_right_logical(idx_i[sl], 2)

        vbufs = (va_v, vb_v)
        vsems = (s_a, s_b)

        def vis_fire(c):
            pltpu.async_copy(
                vf_h.at[idx_i.at[pl.ds(c * CH, CH)]], vbufs[c % 2],
                vsems[c % 2])

        def vis_drain(c):
            pltpu.make_async_copy(
                vf_h.at[idx_i.at[pl.ds(c * CH, CH)]], vbufs[c % 2],
                vsems[c % 2]).wait()
            pltpu.sync_copy(vbufs[c % 2], vf_o.at[pl.ds(base + c * CH, CH)])

        wbufs = (wa_v, wb_v)
        wsems = (s_wa, s_wb)
        jobs = [(gu4_h, idx_u4, 0), (gu4_h, idx_u4, 1),
                (gi4_h, idx_i4, 0), (gi4_h, idx_i4, 1),
                (tu4_h, idx_u4, 0), (tu4_h, idx_u4, 1)]

        def fire_job(j):
            tbl, idx4, h = jobs[j]
            pltpu.async_copy(
                tbl.at[idx4.at[pl.ds(h * HALF, HALF)]], wbufs[j % 2],
                wsems[j % 2])

        def wait_job(j):
            tbl, idx4, h = jobs[j]
            pltpu.make_async_copy(
                tbl.at[idx4.at[pl.ds(h * HALF, HALF)]], wbufs[j % 2],
                wsems[j % 2]).wait()

        iota = lax.iota(jnp.int32, LANES)

        def extract_half(wide, idx, h, dst):
            def body(g, _):
                rloc = iota + g * LANES
                rflat = (rloc + h * HALF) * DG
                off = (idx[pl.ds(h * HALF + g * LANES, LANES)] & 3) * DG
                for k in range(DG):
                    vals = plsc.load_gather(wide, [rloc, off + k])
                    plsc.store_scatter(dst, [rflat + k], vals)
                return 0
            lax.fori_loop(0, ng, body, 0)

        def cf_half(wide, h):
            def body(g, _):
                rloc = iota + g * LANES
                rglob = rloc + h * HALF
                off = (idx_i[pl.ds(h * HALF + g * LANES, LANES)] & 3) * DG
                acc = jnp.zeros((LANES,), jnp.float32)
                rflat = rglob * DG
                for k in range(DG):
                    vu = plsc.load_gather(gu_v, [rflat + k])
                    vi = plsc.load_gather(wide, [rloc, off + k])
                    acc = acc + vu * vi
                cf_v[pl.ds(h * HALF + g * LANES, LANES)] = acc
                return 0
            lax.fori_loop(0, ng, body, 0)

        def process_job(j):
            tbl, idx4, h = jobs[j]
            wide = wbufs[j % 2]
            if j < 2:
                extract_half(wide, idx_u, h, gu_v)
            elif j < 4:
                cf_half(wide, h)
            else:
                extract_half(wide, idx_u, h, tu_v)

        vis_fire(0)
        vis_fire(1)
        fire_job(0)
        fire_job(1)
        vc = 0
        for j in range(6):
            wait_job(j)
            process_job(j)
            if j + 2 < 6:
                fire_job(j + 2)
            for _ in range(nch // 8):
                if vc < nch:
                    vis_drain(vc)
                    if vc + 2 < nch:
                        vis_fire(vc + 2)
                    vc += 1
        while vc < nch:
            vis_drain(vc)
            if vc + 2 < nch:
                vis_fire(vc + 2)
            vc += 1

        pltpu.sync_copy(tu_v, tu_o.at[pl.ds(base * DG, bpw * DG)])
        pltpu.sync_copy(cf_v, cf_o.at[pl.ds(base, bpw)])

    return sc_k(user_ids, item_ids, gamma_user4, gamma_item4, theta_user4,
                visual_features)


def _tc_score(tu_g, vis_g, E_w, cf2d):
    B = tu_g.shape[0]
    DG = tu_g.shape[1]
    DV = vis_g.shape[1]
    BLK = 1024
    nblk = B // BLK

    def body(tu_r, vis_r, ew_r, cf_r, out_r):
        proj = lax.dot_general(
            vis_r[...], ew_r[...], (((1,), (1,)), ((), ())),
            preferred_element_type=jnp.float32,
            precision=lax.Precision.HIGHEST)
        vs = jnp.sum(tu_r[...] * proj, axis=1)
        out_r[...] = cf_r[...] + vs.reshape(BLK // 128, 128)

    out2d = pl.pallas_call(
        body,
        grid=(nblk,),
        in_specs=[
            pl.BlockSpec((BLK, DG), lambda i: (i, 0)),
            pl.BlockSpec((BLK, DV), lambda i: (i, 0)),
            pl.BlockSpec((DG, DV), lambda i: (0, 0)),
            pl.BlockSpec((BLK // 128, 128), lambda i: (i, 0)),
        ],
        out_specs=pl.BlockSpec((BLK // 128, 128), lambda i: (i, 0)),
        out_shape=jax.ShapeDtypeStruct((B // 128, 128), jnp.float32),
    )(tu_g, vis_g, E_w, cf2d)
    return out2d.reshape(B)


def kernel(user_ids, item_ids, gamma_user, gamma_item, theta_user, E_w,
           visual_features):
    gu4 = gamma_user.reshape(-1, 128)
    gi4 = gamma_item.reshape(-1, 128)
    tu4 = theta_user.reshape(-1, 128)
    tu_flat, vis_g, cf = _sc_gather(user_ids, item_ids, gu4, gi4, tu4,
                                    visual_features)
    tu_g = tu_flat.reshape(-1, 32)
    cf2d = cf.reshape(-1, 128)
    return _tc_score(tu_g, vis_g, E_w, cf2d)


# speedup vs baseline: 1.4662x; 1.3613x over previous
import functools

import jax
import jax.numpy as jnp
from jax import lax
from jax.experimental import pallas as pl
from jax.experimental.pallas import tpu as pltpu
from jax.experimental.pallas import tpu_sc as plsc

NC = 2
NS = 16
NW = NC * NS
LANES = 16
DG = 32
BOUT_PAD = 1024


def _sc_visual(item_ids, visual_features):
    B = item_ids.shape[0]
    DV = visual_features.shape[1]
    bpw = B // NW
    CH = 64
    nch = bpw // CH

    @functools.partial(
        pl.kernel,
        out_type=[jax.ShapeDtypeStruct((B, DV), jnp.float32)],
        mesh=plsc.VectorSubcoreMesh(core_axis_name="c", subcore_axis_name="s",
                                    num_cores=NC, num_subcores=NS),
        compiler_params=pltpu.CompilerParams(needs_layout_passes=False),
        scratch_types=[
            pltpu.VMEM((bpw,), jnp.int32),
            pltpu.VMEM((CH, DV), jnp.float32),
            pltpu.VMEM((CH, DV), jnp.float32),
            pltpu.SemaphoreType.DMA,
            pltpu.SemaphoreType.DMA,
        ],
    )
    def sc_v(iid_h, vf_h, vf_o, idx_i, va_v, vb_v, s_a, s_b):
        wid = lax.axis_index("s") * NC + lax.axis_index("c")
        base = wid * bpw
        pltpu.sync_copy(iid_h.at[pl.ds(base, bpw)], idx_i)
        bufs = (va_v, vb_v)
        sems = (s_a, s_b)

        def fire(c):
            pltpu.async_copy(
                vf_h.at[idx_i.at[pl.ds(c * CH, CH)]], bufs[c % 2],
                sems[c % 2])

        fire(0)
        fire(1)
        for c in range(nch):
            pltpu.make_async_copy(
                vf_h.at[idx_i.at[pl.ds(c * CH, CH)]], bufs[c % 2],
                sems[c % 2]).wait()
            pltpu.sync_copy(bufs[c % 2], vf_o.at[pl.ds(base + c * CH, CH)])
            if c + 2 < nch:
                fire(c + 2)

    return sc_v(item_ids, visual_features)[0]


def _sc_item(item_ids, gamma_item4):
    B = item_ids.shape[0]
    bpw = B // NW
    HALF = bpw // 2
    ng = HALF // LANES

    @functools.partial(
        pl.kernel,
        out_type=[jax.ShapeDtypeStruct((B * DG,), jnp.float32)],
        mesh=plsc.VectorSubcoreMesh(core_axis_name="c", subcore_axis_name="s",
                                    num_cores=NC, num_subcores=NS),
        compiler_params=pltpu.CompilerParams(needs_layout_passes=False),
        scratch_types=[
            pltpu.VMEM((bpw,), jnp.int32),
            pltpu.VMEM((bpw,), jnp.int32),
            pltpu.VMEM((HALF, 128), jnp.float32),
            pltpu.VMEM((HALF, 128), jnp.float32),
            pltpu.VMEM((bpw * DG,), jnp.float32),
            pltpu.SemaphoreType.DMA,
            pltpu.SemaphoreType.DMA,
        ],
    )
    def sc_n(iid_h, gi4_h, gi_o, idx_i, idx_i4, wa_v, wb_v, gi_v, s_wa, s_wb):
        wid = lax.axis_index("s") * NC + lax.axis_index("c")
        base = wid * bpw
        pltpu.sync_copy(iid_h.at[pl.ds(base, bpw)], idx_i)
        for j in range(bpw // LANES):
            sl = pl.ds(j * LANES, LANES)
            idx_i4[sl] = lax.shift_right_logical(idx_i[sl], 2)

        wbufs = (wa_v, wb_v)
        wsems = (s_wa, s_wb)

        def fire_job(h):
            pltpu.async_copy(
                gi4_h.at[idx_i4.at[pl.ds(h * HALF, HALF)]], wbufs[h % 2],
                wsems[h % 2])

        iota = lax.iota(jnp.int32, LANES)

        def extract_half(h):
            wide = wbufs[h % 2]

            def body(g, _):
                rloc = iota + g * LANES
                rflat = (rloc + h * HALF) * DG
                off = (idx_i[pl.ds(h * HALF + g * LANES, LANES)] & 3) * DG
                for k in range(DG):
                    vals = plsc.load_gather(wide, [rloc, off + k])
                    plsc.store_scatter(gi_v, [rflat + k], vals)
                return 0
            lax.fori_loop(0, ng, body, 0)

        fire_job(0)
        fire_job(1)
        for h in range(2):
            pltpu.make_async_copy(
                gi4_h.at[idx_i4.at[pl.ds(h * HALF, HALF)]], wbufs[h % 2],
                wsems[h % 2]).wait()
            extract_half(h)

        pltpu.sync_copy(gi_v, gi_o.at[pl.ds(base * DG, bpw * DG)])

    return sc_n(item_ids, gamma_item4)[0]


def _sc_user_sweep(user_ids, guT, tuT, gu_tail4, tu_tail4):
    B = user_ids.shape[0]
    NU = guT.shape[1]
    NBLK = NU // 128
    TAIL = NU - NBLK * 128
    base_blocks = NBLK // NW
    extra = NBLK - base_blocks * NW
    max_blocks = base_blocks + (1 if extra else 0)
    NITER = (max_blocks + 3) // 4
    HIT_CAP = 1024
    NGRP_B = B // LANES
    NGRP_H = HIT_CAP // LANES
    BOUT = B + BOUT_PAD

    @functools.partial(
        pl.kernel,
        out_type=[
            jax.ShapeDtypeStruct((BOUT, 128), jnp.float32),
            jax.ShapeDtypeStruct((BOUT, 128), jnp.float32),
        ],
        mesh=plsc.VectorSubcoreMesh(core_axis_name="c", subcore_axis_name="s",
                                    num_cores=NC, num_subcores=NS),
        compiler_params=pltpu.CompilerParams(needs_layout_passes=False),
        scratch_types=[
            pltpu.VMEM((B,), jnp.int32),
            pltpu.VMEM((HIT_CAP,), jnp.int32),
            pltpu.VMEM((HIT_CAP,), jnp.int32),
            pltpu.VMEM((4, DG, 128), jnp.float32),
            pltpu.VMEM((4, DG, 128), jnp.float32),
            pltpu.VMEM((4, LANES, 128), jnp.float32),
            pltpu.VMEM((4, LANES, 128), jnp.float32),
            pltpu.VMEM((gu_tail4.shape[0], 128), jnp.float32),
            pltpu.VMEM((gu_tail4.shape[0], 128), jnp.float32),
        ] + [pltpu.SemaphoreType.DMA] * 16,
    )
    def sc_u(uid_h, guT_h, tuT_h, gt4_h, tt4_h, gu_o, tu_o,
             ids_v, hit_u, hit_p, gub_v, tub_v, sgu_v, stu_v, tg_v, tt_v,
             *sems):
        s_gb = sems[0:4]
        s_tb = sems[4:8]
        s_gs = sems[8:12]
        s_ts = sems[12:16]
        wid = lax.axis_index("s") * NC + lax.axis_index("c")
        is_last = wid == NW - 1
        nblk_w = base_blocks + jnp.where(wid < extra, 1, 0)
        blk0 = wid * base_blocks + jnp.minimum(wid, extra)
        dump = B + wid * LANES
        iota = lax.iota(jnp.int32, LANES)

        pltpu.sync_copy(uid_h, ids_v)

        blk_hi = blk0 + nblk_w + jnp.where(is_last, 1, 0)

        def scan_body(g, cnt):
            u = ids_v[pl.ds(g * LANES, LANES)]
            blk = lax.shift_right_logical(u, 7)
            m = (blk >= blk0) & (blk < blk_hi)
            plsc.store_compressed(hit_u.at[pl.ds(cnt, LANES)], u, mask=m)
            plsc.store_compressed(hit_p.at[pl.ds(cnt, LANES)],
                                  iota + g * LANES, mask=m)
            npop = plsc.all_reduce_population_count(m)
            return cnt + npop[0]

        cnt = lax.fori_loop(0, NGRP_B, scan_body, jnp.int32(0))
        hit_u[pl.ds(cnt, LANES)] = jnp.full((LANES,), 1 << 30, jnp.int32)
        hit_p[pl.ds(cnt, LANES)] = dump + iota
        hit_u[pl.ds(cnt + LANES, LANES)] = jnp.full((LANES,), 1 << 30,
                                                    jnp.int32)
        hit_p[pl.ds(cnt + LANES, LANES)] = dump + iota
        ngrp_hit = lax.shift_right_logical(cnt + LANES - 1, 4) + 1

        for j in range(4):
            pltpu.async_copy(sgu_v.at[j], gu_o.at[dump + iota], s_gs[j])
            pltpu.async_copy(stu_v.at[j], tu_o.at[dump + iota], s_ts[j])

        def extract_block(gub, tub, blk, j):
            def grp_body(g, _):
                u = hit_u[pl.ds(g * LANES, LANES)]
                m = lax.shift_right_logical(u, 7) == blk

                @pl.when(plsc.all_reduce_population_count(m)[0] > 0)
                def _():
                    pos = hit_p[pl.ds(g * LANES, LANES)]
                    col = u & 127
                    posq = jnp.where(m, pos, dump + iota)
                    pltpu.make_async_copy(
                        sgu_v.at[j], gu_o.at[posq], s_gs[j]).wait()
                    pltpu.make_async_copy(
                        stu_v.at[j], tu_o.at[posq], s_ts[j]).wait()
                    for k in range(DG):
                        kv = jnp.full((LANES,), k, jnp.int32)
                        vg = plsc.load_gather(gub, [kv, col])
                        plsc.store_scatter(sgu_v.at[j], [iota, kv], vg)
                        vt = plsc.load_gather(tub, [kv, col])
                        plsc.store_scatter(stu_v.at[j], [iota, kv], vt)
                    pltpu.async_copy(sgu_v.at[j], gu_o.at[posq], s_gs[j])
                    pltpu.async_copy(stu_v.at[j], tu_o.at[posq], s_ts[j])
                return 0
            lax.fori_loop(0, ngrp_hit, grp_body, 0)

        def fire_block(blk, j):
            ws = pl.multiple_of(blk * 128, 128)
            pltpu.async_copy(guT_h.at[:, pl.ds(ws, 128)], gub_v.at[j],
                             s_gb[j])
            pltpu.async_copy(tuT_h.at[:, pl.ds(ws, 128)], tub_v.at[j],
                             s_tb[j])

        def wait_block(blk, j):
            ws = pl.multiple_of(blk * 128, 128)
            pltpu.make_async_copy(
                guT_h.at[:, pl.ds(ws, 128)], gub_v.at[j], s_gb[j]).wait()
            pltpu.make_async_copy(
                tuT_h.at[:, pl.ds(ws, 128)], tub_v.at[j], s_tb[j]).wait()

        for j in range(4):
            @pl.when(j < nblk_w)
            def _(j=j):
                fire_block(blk0 + j, j)

        def sweep_body(i, _):
            for j in range(4):
                bi = i * 4 + j

                @pl.when(bi < nblk_w)
                def _(bi=bi, j=j):
                    blk = blk0 + bi
                    wait_block(blk, j)
                    extract_block(gub_v.at[j], tub_v.at[j], blk, j)

                    @pl.when(bi + 4 < nblk_w)
                    def _(bi=bi, j=j, blk=blk):
                        fire_block(blk + 4, j)
            return 0

        lax.fori_loop(0, NITER, sweep_body, 0)

        if TAIL:
            @pl.when(is_last)
            def _():
                pltpu.sync_copy(gt4_h, tg_v)
                pltpu.sync_copy(tt4_h, tt_v)

                def tail_grp(g, _):
                    u = hit_u[pl.ds(g * LANES, LANES)]
                    m = lax.shift_right_logical(u, 7) == NBLK

                    @pl.when(plsc.all_reduce_population_count(m)[0] > 0)
                    def _():
                        pos = hit_p[pl.ds(g * LANES, LANES)]
                        ul = u - NBLK * 128
                        line = lax.shift_right_logical(ul, 2)
                        off = (ul & 3) * DG
                        posq = jnp.where(m, pos, dump + iota)
                        pltpu.make_async_copy(
                            sgu_v.at[0], gu_o.at[posq], s_gs[0]).wait()
                        pltpu.make_async_copy(
                            stu_v.at[0], tu_o.at[posq], s_ts[0]).wait()
                        for k in range(DG):
                            kv = jnp.full((LANES,), k, jnp.int32)
                            vg = plsc.load_gather(tg_v, [line, off + k])
                            plsc.store_scatter(sgu_v.at[0], [iota, kv], vg)
                            vt = plsc.load_gather(tt_v, [line, off + k])
                            plsc.store_scatter(stu_v.at[0], [iota, kv], vt)
                        pltpu.async_copy(sgu_v.at[0], gu_o.at[posq],
                                         s_gs[0])
                        pltpu.async_copy(stu_v.at[0], tu_o.at[posq],
                                         s_ts[0])
                    return 0

                lax.fori_loop(0, ngrp_hit, tail_grp, 0)

        for j in range(4):
            pltpu.make_async_copy(sgu_v.at[j], gu_o.at[dump + iota],
                                  s_gs[j]).wait()
            pltpu.make_async_copy(stu_v.at[j], tu_o.at[dump + iota],
                                  s_ts[j]).wait()

    return sc_u(user_ids, guT, tuT, gu_tail4, tu_tail4)


def _tc_score(gu_pad, tu_pad, gi_g, vis_g, E_w):
    B = vis_g.shape[0]
    DV = vis_g.shape[1]
    BLK = 1024
    nblk = B // BLK

    def body(gu_r, tu_r, gi_r, vis_r, ew_r, out_r):
        proj = lax.dot_general(
            vis_r[...], ew_r[...], (((1,), (1,)), ((), ())),
            preferred_element_type=jnp.float32,
            precision=lax.Precision.HIGHEST)
        cf = jnp.sum(gu_r[:, :DG] * gi_r[...], axis=1)
        vs = jnp.sum(tu_r[:, :DG] * proj, axis=1)
        out_r[...] = (cf + vs).reshape(BLK // 128, 128)

    out2d = pl.pallas_call(
        body,
        grid=(nblk,),
        in_specs=[
            pl.BlockSpec((BLK, 128), lambda i: (i, 0)),
            pl.BlockSpec((BLK, 128), lambda i: (i, 0)),
            pl.BlockSpec((BLK, DG), lambda i: (i, 0)),
            pl.BlockSpec((BLK, DV), lambda i: (i, 0)),
            pl.BlockSpec((DG, DV), lambda i: (0, 0)),
        ],
        out_specs=pl.BlockSpec((BLK // 128, 128), lambda i: (i, 0)),
        out_shape=jax.ShapeDtypeStruct((B // 128, 128), jnp.float32),
    )(gu_pad, tu_pad, gi_g, vis_g, E_w)
    return out2d.reshape(B)


def kernel(user_ids, item_ids, gamma_user, gamma_item, theta_user, E_w,
           visual_features):
    gi4 = gamma_item.reshape(-1, 128)
    vis_g = _sc_visual(item_ids, visual_features)
    gi_flat = _sc_item(item_ids, gi4)
    nu = gamma_user.shape[0]
    cut = (nu // 128) * 128
    gu_pad, tu_pad = _sc_user_sweep(
        user_ids, gamma_user.T, theta_user.T,
        gamma_user[cut:].reshape(-1, 128), theta_user[cut:].reshape(-1, 128))
    gi_g = gi_flat.reshape(-1, DG)
    return _tc_score(gu_pad, tu_pad, gi_g, vis_g, E_w)
